# Initial kernel scaffold; baseline (speedup 1.0000x reference)
import jax
import jax.numpy as jnp
from jax.experimental import pallas as pl


def kernel(c1, c2, c3, c4, proj_w_0, proj_w_1, proj_w_2, proj_w_3, proj_b_0, proj_b_1, proj_b_2, proj_b_3, fuse_w, bn_scale, bn_shift, pred_w, pred_b, interp_h1, interp_w1, interp_h2, interp_w2, interp_h3, interp_w3):
    raise NotImplementedError("write your pallas kernel here")


# speedup vs baseline: 4.5492x; 4.5492x over previous
import jax
import jax.numpy as jnp
from jax.experimental import pallas as pl
from jax.experimental.pallas import tpu as pltpu

_TH = 2


def _head_body(c1_ref, x2_ref, x3_ref, x4_ref,
               w0_ref, w1_ref, w2_ref, w3_ref,
               ah1_ref, aw1_ref, ah2_ref, aw2_ref, ah3_ref, aw3_ref,
               sh_ref, predT_ref, pb_ref, o_ref):
    f32 = jnp.float32
    E = w0_ref.shape[0]
    Ht, W1 = o_ref.shape[2], o_ref.shape[3]

    dn = (((1,), (0,)), ((), ()))
    s = jax.lax.dot_general(w0_ref[...], c1_ref[0], dn,
                            preferred_element_type=f32)
    s = s.reshape(E * Ht, W1)

    for x_ref, w_ref, ah_ref, aw_ref in (
            (x2_ref, w1_ref, ah1_ref, aw1_ref),
            (x3_ref, w2_ref, ah2_ref, aw2_ref),
            (x4_ref, w3_ref, ah3_ref, aw3_ref)):
        w = x_ref.shape[2]
        h = x_ref.shape[3]
        p = jax.lax.dot_general(w_ref[...], x_ref[0], dn,
                                preferred_element_type=f32)
        q = jnp.dot(p.reshape(E * w, h), ah_ref[0],
                    preferred_element_type=f32)
        qt = jnp.transpose(q.reshape(E, w, Ht), (0, 2, 1))
        u = jnp.dot(qt.reshape(E * Ht, w), aw_ref[...],
                    preferred_element_type=f32)
        s = s + u

    y = jnp.maximum(s + sh_ref[...].reshape(E * Ht, 1), 0.0)
    o = jax.lax.dot_general(predT_ref[...], y.reshape(E, Ht, W1), dn,
                            preferred_element_type=f32)
    o_ref[0] = o + pb_ref[...]


def kernel(c1, c2, c3, c4,
           proj_w_0, proj_w_1, proj_w_2, proj_w_3,
           proj_b_0, proj_b_1, proj_b_2, proj_b_3,
           fuse_w, bn_scale, bn_shift, pred_w, pred_b,
           interp_h1, interp_w1, interp_h2, interp_w2, interp_h3, interp_w3):
    n, C1, H1, W1 = c1.shape
    E = fuse_w.shape[1]
    ncls = pred_w.shape[1]
    f32 = jnp.float32
    Ht = H1 // _TH

    proj_w = [proj_w_0, proj_w_1, proj_w_2, proj_w_3]
    proj_b = [proj_b_0, proj_b_1, proj_b_2, proj_b_3]
    wfT = []
    bias = jnp.zeros((1, E), f32)
    for i in range(4):
        blk = fuse_w[(3 - i) * E:(4 - i) * E, :]
        wf = proj_w[i] @ blk
        wfT.append(jnp.asarray((wf * bn_scale).T, f32))
        bias = bias + proj_b[i] @ blk
    shift = (bn_shift + bias * bn_scale).reshape(E, 1, 1)
    shift3 = jnp.broadcast_to(shift, (E, H1, 1))
    predT = jnp.asarray(pred_w.T, f32)
    pb3 = jnp.broadcast_to(pred_b.reshape(ncls, 1, 1), (ncls, H1, 1))

    x2 = jnp.transpose(c2, (0, 1, 3, 2))
    x3 = jnp.transpose(c3, (0, 1, 3, 2))
    x4 = jnp.transpose(c4, (0, 1, 3, 2))
    tsplit = lambda a: jnp.transpose(
        a.T.reshape(a.shape[1], _TH, Ht), (1, 0, 2))
    ah1, aw1 = tsplit(interp_h1), interp_w1.T
    ah2, aw2 = tsplit(interp_h2), interp_w2.T
    ah3, aw3 = tsplit(interp_h3), interp_w3.T

    _, C2, w2, h2 = x2.shape
    _, C3, w3, h3 = x3.shape
    _, C4, w4, h4 = x4.shape

    full = lambda shape: pl.BlockSpec(shape, lambda b, t: (0,) * len(shape))
    coarse = lambda shape: pl.BlockSpec(shape, lambda b, t: (b, 0, 0, 0))

    out = pl.pallas_call(
        _head_body,
        out_shape=jax.ShapeDtypeStruct((n, ncls, H1, W1), f32),
        grid=(n, _TH),
        in_specs=[
            pl.BlockSpec((1, C1, Ht, W1), lambda b, t: (b, 0, t, 0)),
            coarse((1, C2, w2, h2)),
            coarse((1, C3, w3, h3)),
            coarse((1, C4, w4, h4)),
            full((E, C1)), full((E, C2)), full((E, C3)), full((E, C4)),
            pl.BlockSpec((1, h2, Ht), lambda b, t: (t, 0, 0)), full((w2, W1)),
            pl.BlockSpec((1, h3, Ht), lambda b, t: (t, 0, 0)), full((w3, W1)),
            pl.BlockSpec((1, h4, Ht), lambda b, t: (t, 0, 0)), full((w4, W1)),
            pl.BlockSpec((E, Ht, 1), lambda b, t: (0, t, 0)),
            full((ncls, E)),
            pl.BlockSpec((ncls, Ht, 1), lambda b, t: (0, t, 0)),
        ],
        out_specs=pl.BlockSpec((1, ncls, Ht, W1), lambda b, t: (b, 0, t, 0)),
        compiler_params=pltpu.CompilerParams(
            dimension_semantics=("parallel", "parallel")),
    )(c1, x2, x3, x4,
      wfT[0], wfT[1], wfT[2], wfT[3],
      ah1, aw1, ah2, aw2, ah3, aw3,
      shift3, predT, pb3)
    return out


# speedup vs baseline: 5.0744x; 1.1154x over previous
import jax
import jax.numpy as jnp
from jax.experimental import pallas as pl
from jax.experimental.pallas import tpu as pltpu

_TH = 2


def _head_body(c1_ref, x2_ref, x3_ref, x4_ref,
               w0_ref, w1_ref, w2_ref, w3_ref,
               ah1_ref, ah2_ref, ah3_ref, aw_ref,
               sh_ref, predT_ref, pb_ref, o_ref):
    f32 = jnp.float32
    E = w0_ref.shape[0]
    Ht, W1 = o_ref.shape[2], o_ref.shape[3]

    dn = (((1,), (0,)), ((), ()))
    s = jax.lax.dot_general(w0_ref[...], c1_ref[0], dn,
                            preferred_element_type=f32)

    qs = []
    for x_ref, w_ref, ah_ref in (
            (x2_ref, w1_ref, ah1_ref),
            (x3_ref, w2_ref, ah2_ref),
            (x4_ref, w3_ref, ah3_ref)):
        w = x_ref.shape[2]
        h = x_ref.shape[3]
        p = jax.lax.dot_general(w_ref[...], x_ref[0], dn,
                                preferred_element_type=f32)
        q = jnp.dot(p.reshape(E * w, h), ah_ref[0],
                    preferred_element_type=f32)
        qs.append(q.reshape(E, w, Ht))

    qcat = jnp.concatenate(qs, axis=1)
    dnm = (((1,), (0,)), ((), ()))
    u = jax.lax.dot_general(qcat, aw_ref[...], dnm,
                            preferred_element_type=f32)
    s = s + u

    y = jnp.maximum(s + sh_ref[...], 0.0)
    o = jax.lax.dot_general(predT_ref[...], y, dn,
                            preferred_element_type=f32)
    o_ref[0] = o + pb_ref[...]


def kernel(c1, c2, c3, c4,
           proj_w_0, proj_w_1, proj_w_2, proj_w_3,
           proj_b_0, proj_b_1, proj_b_2, proj_b_3,
           fuse_w, bn_scale, bn_shift, pred_w, pred_b,
           interp_h1, interp_w1, interp_h2, interp_w2, interp_h3, interp_w3):
    n, C1, H1, W1 = c1.shape
    E = fuse_w.shape[1]
    ncls = pred_w.shape[1]
    f32 = jnp.float32
    Ht = H1 // _TH

    proj_w = [proj_w_0, proj_w_1, proj_w_2, proj_w_3]
    proj_b = [proj_b_0, proj_b_1, proj_b_2, proj_b_3]
    wfT = []
    bias = jnp.zeros((1, E), f32)
    for i in range(4):
        blk = fuse_w[(3 - i) * E:(4 - i) * E, :]
        wf = proj_w[i] @ blk
        wfT.append(jnp.asarray((wf * bn_scale).T, f32))
        bias = bias + proj_b[i] @ blk
    shift = (bn_shift + bias * bn_scale).reshape(E, 1, 1)
    shift3 = jnp.broadcast_to(shift, (E, H1, 1))
    predT = jnp.asarray(pred_w.T, f32)
    pb3 = jnp.broadcast_to(pred_b.reshape(ncls, 1, 1), (ncls, H1, 1))

    x2 = jnp.transpose(c2, (0, 1, 3, 2))
    x3 = jnp.transpose(c3, (0, 1, 3, 2))
    x4 = jnp.transpose(c4, (0, 1, 3, 2))
    tsplit = lambda a: jnp.transpose(
        a.T.reshape(a.shape[1], _TH, Ht), (1, 0, 2))
    ah1, ah2, ah3 = tsplit(interp_h1), tsplit(interp_h2), tsplit(interp_h3)
    awcat = jnp.concatenate(
        [interp_w1.T, interp_w2.T, interp_w3.T], axis=0)

    _, C2, w2, h2 = x2.shape
    _, C3, w3, h3 = x3.shape
    _, C4, w4, h4 = x4.shape

    full = lambda shape: pl.BlockSpec(shape, lambda b, t: (0,) * len(shape))
    coarse = lambda shape: pl.BlockSpec(shape, lambda b, t: (b, 0, 0, 0))

    out = pl.pallas_call(
        _head_body,
        out_shape=jax.ShapeDtypeStruct((n, ncls, H1, W1), f32),
        grid=(n, _TH),
        in_specs=[
            pl.BlockSpec((1, C1, Ht, W1), lambda b, t: (b, 0, t, 0)),
            coarse((1, C2, w2, h2)),
            coarse((1, C3, w3, h3)),
            coarse((1, C4, w4, h4)),
            full((E, C1)), full((E, C2)), full((E, C3)), full((E, C4)),
            pl.BlockSpec((1, h2, Ht), lambda b, t: (t, 0, 0)),
            pl.BlockSpec((1, h3, Ht), lambda b, t: (t, 0, 0)),
            pl.BlockSpec((1, h4, Ht), lambda b, t: (t, 0, 0)),
            full((w2 + w3 + w4, W1)),
            pl.BlockSpec((E, Ht, 1), lambda b, t: (0, t, 0)),
            full((ncls, E)),
            pl.BlockSpec((ncls, Ht, 1), lambda b, t: (0, t, 0)),
        ],
        out_specs=pl.BlockSpec((1, ncls, Ht, W1), lambda b, t: (b, 0, t, 0)),
        compiler_params=pltpu.CompilerParams(
            dimension_semantics=("parallel", "parallel")),
    )(c1, x2, x3, x4,
      wfT[0], wfT[1], wfT[2], wfT[3],
      ah1, ah2, ah3, awcat,
      shift3, predT, pb3)
    return out


# speedup vs baseline: 5.4119x; 1.0665x over previous
import jax
import jax.numpy as jnp
from jax.experimental import pallas as pl
from jax.experimental.pallas import tpu as pltpu

_TH = 2


def _head_body(c1_ref, x2_ref, x3_ref, x4_ref,
               w0_ref, w1_ref, w2_ref, w3_ref,
               ah1_ref, ah2_ref, ah3_ref, aw_ref,
               sh_ref, predT_ref, pb_ref, o_ref):
    f32 = jnp.float32
    bf16 = jnp.bfloat16
    E = w0_ref.shape[0]
    Ht, W1 = o_ref.shape[2], o_ref.shape[3]
    dn = (((1,), (0,)), ((), ()))

    qs = []
    for x_ref, w_ref, ah_ref in (
            (x2_ref, w1_ref, ah1_ref),
            (x3_ref, w2_ref, ah2_ref),
            (x4_ref, w3_ref, ah3_ref)):
        w = x_ref.shape[2]
        h = x_ref.shape[3]
        p = jax.lax.dot_general(w_ref[...], x_ref[0], dn,
                                preferred_element_type=f32)
        q = jnp.dot(p.reshape(E * w, h).astype(bf16), ah_ref[0],
                    preferred_element_type=f32)
        qs.append(q.reshape(E, w, Ht).astype(bf16))

    qcat = jnp.concatenate(qs, axis=1)
    u = jax.lax.dot_general(qcat, aw_ref[...], dn,
                            preferred_element_type=f32)

    s = jax.lax.dot_general(w0_ref[...], c1_ref[0], dn,
                            preferred_element_type=f32)
    y = jnp.maximum(s + u + sh_ref[...], 0.0).astype(bf16)
    o = jax.lax.dot_general(predT_ref[...], y, dn,
                            preferred_element_type=f32)
    o_ref[0] = o + pb_ref[...]


def kernel(c1, c2, c3, c4,
           proj_w_0, proj_w_1, proj_w_2, proj_w_3,
           proj_b_0, proj_b_1, proj_b_2, proj_b_3,
           fuse_w, bn_scale, bn_shift, pred_w, pred_b,
           interp_h1, interp_w1, interp_h2, interp_w2, interp_h3, interp_w3):
    n, C1, H1, W1 = c1.shape
    E = fuse_w.shape[1]
    ncls = pred_w.shape[1]
    f32 = jnp.float32
    Ht = H1 // _TH

    proj_w = [proj_w_0, proj_w_1, proj_w_2, proj_w_3]
    proj_b = [proj_b_0, proj_b_1, proj_b_2, proj_b_3]
    wfT = []
    bias = jnp.zeros((1, E), f32)
    for i in range(4):
        blk = fuse_w[(3 - i) * E:(4 - i) * E, :]
        wf = proj_w[i] @ blk
        wfT.append(jnp.asarray((wf * bn_scale).T, jnp.bfloat16))
        bias = bias + proj_b[i] @ blk
    shift = (bn_shift + bias * bn_scale).reshape(E, 1, 1)
    shift3 = jnp.broadcast_to(shift, (E, H1, 1))
    predT = jnp.asarray(pred_w.T, jnp.bfloat16)
    pb3 = jnp.broadcast_to(pred_b.reshape(ncls, 1, 1), (ncls, H1, 1))

    bf16 = jnp.bfloat16
    c1 = c1.astype(bf16)
    x2 = jnp.transpose(c2, (0, 1, 3, 2)).astype(bf16)
    x3 = jnp.transpose(c3, (0, 1, 3, 2)).astype(bf16)
    x4 = jnp.transpose(c4, (0, 1, 3, 2)).astype(bf16)
    tsplit = lambda a: jnp.transpose(
        a.T.reshape(a.shape[1], _TH, Ht), (1, 0, 2)).astype(bf16)
    ah1, ah2, ah3 = tsplit(interp_h1), tsplit(interp_h2), tsplit(interp_h3)
    awcat = jnp.concatenate(
        [interp_w1.T, interp_w2.T, interp_w3.T], axis=0).astype(bf16)

    _, C2, w2, h2 = x2.shape
    _, C3, w3, h3 = x3.shape
    _, C4, w4, h4 = x4.shape

    full = lambda shape: pl.BlockSpec(shape, lambda b, t: (0,) * len(shape))
    coarse = lambda shape: pl.BlockSpec(shape, lambda b, t: (b, 0, 0, 0))

    out = pl.pallas_call(
        _head_body,
        out_shape=jax.ShapeDtypeStruct((n, ncls, H1, W1), f32),
        grid=(n, _TH),
        in_specs=[
            pl.BlockSpec((1, C1, Ht, W1), lambda b, t: (b, 0, t, 0)),
            coarse((1, C2, w2, h2)),
            coarse((1, C3, w3, h3)),
            coarse((1, C4, w4, h4)),
            full((E, C1)), full((E, C2)), full((E, C3)), full((E, C4)),
            pl.BlockSpec((1, h2, Ht), lambda b, t: (t, 0, 0)),
            pl.BlockSpec((1, h3, Ht), lambda b, t: (t, 0, 0)),
            pl.BlockSpec((1, h4, Ht), lambda b, t: (t, 0, 0)),
            full((w2 + w3 + w4, W1)),
            pl.BlockSpec((E, Ht, 1), lambda b, t: (0, t, 0)),
            full((ncls, E)),
            pl.BlockSpec((ncls, Ht, 1), lambda b, t: (0, t, 0)),
        ],
        out_specs=pl.BlockSpec((1, ncls, Ht, W1), lambda b, t: (b, 0, t, 0)),
        compiler_params=pltpu.CompilerParams(
            dimension_semantics=("parallel", "parallel")),
    )(c1, x2, x3, x4,
      wfT[0], wfT[1], wfT[2], wfT[3],
      ah1, ah2, ah3, awcat,
      shift3, predT, pb3)
    return out


# speedup vs baseline: 5.9405x; 1.0977x over previous
import jax
import jax.numpy as jnp
from jax.experimental import pallas as pl
from jax.experimental.pallas import tpu as pltpu

_TH = 1


def _head_body(c1_ref, x2_ref, x3_ref, x4_ref,
               w0_ref, w1_ref, w2_ref, w3_ref,
               ah1_ref, aw1_ref, ah2_ref, ah3_ref, aw34_ref,
               sh_ref, predT_ref, pb_ref, o_ref):
    f32 = jnp.float32
    bf16 = jnp.bfloat16
    E = w0_ref.shape[0]
    Ht, W1 = o_ref.shape[2], o_ref.shape[3]
    dn = (((1,), (0,)), ((), ()))

    C2, w2, h2 = x2_ref.shape[1], x2_ref.shape[2], x2_ref.shape[3]
    xh = jnp.dot(x2_ref[0].reshape(C2 * w2, h2), ah1_ref[0],
                 preferred_element_type=f32)
    xt = jnp.transpose(xh.reshape(C2, w2, Ht), (0, 2, 1))
    xw = jnp.dot(xt.reshape(C2 * Ht, w2).astype(bf16), aw1_ref[...],
                 preferred_element_type=f32)
    u2 = jax.lax.dot_general(w1_ref[...], xw.reshape(C2, Ht, W1).astype(bf16),
                             dn, preferred_element_type=f32)

    qs = []
    for x_ref, w_ref, ah_ref in ((x3_ref, w2_ref, ah2_ref),
                                 (x4_ref, w3_ref, ah3_ref)):
        w = x_ref.shape[2]
        h = x_ref.shape[3]
        p = jax.lax.dot_general(w_ref[...], x_ref[0], dn,
                                preferred_element_type=f32)
        q = jnp.dot(p.reshape(E * w, h).astype(bf16), ah_ref[0],
                    preferred_element_type=f32)
        qs.append(q.reshape(E, w, Ht).astype(bf16))
    qcat = jnp.concatenate(qs, axis=1)
    u34 = jax.lax.dot_general(qcat, aw34_ref[...], dn,
                              preferred_element_type=f32)

    s = jax.lax.dot_general(w0_ref[...], c1_ref[0], dn,
                            preferred_element_type=f32)
    y = jnp.maximum(s + u2 + u34 + sh_ref[...], 0.0).astype(bf16)
    o = jax.lax.dot_general(predT_ref[...], y, dn,
                            preferred_element_type=f32)
    o_ref[0] = o + pb_ref[...]


def kernel(c1, c2, c3, c4,
           proj_w_0, proj_w_1, proj_w_2, proj_w_3,
           proj_b_0, proj_b_1, proj_b_2, proj_b_3,
           fuse_w, bn_scale, bn_shift, pred_w, pred_b,
           interp_h1, interp_w1, interp_h2, interp_w2, interp_h3, interp_w3):
    n, C1, H1, W1 = c1.shape
    E = fuse_w.shape[1]
    ncls = pred_w.shape[1]
    f32 = jnp.float32
    bf16 = jnp.bfloat16
    Ht = H1 // _TH

    proj_w = [proj_w_0, proj_w_1, proj_w_2, proj_w_3]
    proj_b = [proj_b_0, proj_b_1, proj_b_2, proj_b_3]
    wfT = []
    bias = jnp.zeros((1, E), f32)
    for i in range(4):
        blk = fuse_w[(3 - i) * E:(4 - i) * E, :]
        wf = proj_w[i] @ blk
        wfT.append(jnp.asarray((wf * bn_scale).T, bf16))
        bias = bias + proj_b[i] @ blk
    shift = (bn_shift + bias * bn_scale).reshape(E, 1, 1)
    shift3 = jnp.broadcast_to(shift, (E, H1, 1))
    predT = jnp.asarray(pred_w.T, bf16)
    pb3 = jnp.broadcast_to(pred_b.reshape(ncls, 1, 1), (ncls, H1, 1))

    c1 = c1.astype(bf16)
    x2 = jnp.transpose(c2, (0, 1, 3, 2)).astype(bf16)
    x3 = jnp.transpose(c3, (0, 1, 3, 2)).astype(bf16)
    x4 = jnp.transpose(c4, (0, 1, 3, 2)).astype(bf16)
    tsplit = lambda a: jnp.transpose(
        a.T.reshape(a.shape[1], _TH, Ht), (1, 0, 2)).astype(bf16)
    ah1, ah2, ah3 = tsplit(interp_h1), tsplit(interp_h2), tsplit(interp_h3)
    aw1 = jnp.asarray(interp_w1.T, bf16)
    aw34 = jnp.concatenate(
        [interp_w2.T, interp_w3.T], axis=0).astype(bf16)

    _, C2, w2, h2 = x2.shape
    _, C3, w3, h3 = x3.shape
    _, C4, w4, h4 = x4.shape

    full = lambda shape: pl.BlockSpec(shape, lambda b, t: (0,) * len(shape))
    coarse = lambda shape: pl.BlockSpec(shape, lambda b, t: (b, 0, 0, 0))

    out = pl.pallas_call(
        _head_body,
        out_shape=jax.ShapeDtypeStruct((n, ncls, H1, W1), f32),
        grid=(n, _TH),
        in_specs=[
            pl.BlockSpec((1, C1, Ht, W1), lambda b, t: (b, 0, t, 0)),
            coarse((1, C2, w2, h2)),
            coarse((1, C3, w3, h3)),
            coarse((1, C4, w4, h4)),
            full((E, C1)), full((E, C2)), full((E, C3)), full((E, C4)),
            pl.BlockSpec((1, h2, Ht), lambda b, t: (t, 0, 0)),
            full((w2, W1)),
            pl.BlockSpec((1, h3, Ht), lambda b, t: (t, 0, 0)),
            pl.BlockSpec((1, h4, Ht), lambda b, t: (t, 0, 0)),
            full((w3 + w4, W1)),
            pl.BlockSpec((E, Ht, 1), lambda b, t: (0, t, 0)),
            full((ncls, E)),
            pl.BlockSpec((ncls, Ht, 1), lambda b, t: (0, t, 0)),
        ],
        out_specs=pl.BlockSpec((1, ncls, Ht, W1), lambda b, t: (b, 0, t, 0)),
        compiler_params=pltpu.CompilerParams(
            dimension_semantics=("parallel", "parallel")),
    )(c1, x2, x3, x4,
      wfT[0], wfT[1], wfT[2], wfT[3],
      ah1, aw1, ah2, ah3, aw34,
      shift3, predT, pb3)
    return out
